# baseline (speedup 1.0000x reference)
import functools

import jax
import jax.numpy as jnp
from jax import lax
from jax.experimental import pallas as pl
from jax.experimental.pallas import tpu as pltpu
from jax.experimental.pallas import tpu_sc as plsc

NC = 2
NS = 16
NW = NC * NS

V = 1440
D = 64
B = 4096 * 200

B_SC = 409600
B_TC = B - B_SC

SUB = 128
CH = 640
B_PER_W = B_SC // NW
N_CHUNKS = B_PER_W // CH
NPAIR = N_CHUNKS // 2

BLK = 512
N_BLK = B_TC // BLK

_mesh = plsc.VectorSubcoreMesh(core_axis_name="c", subcore_axis_name="s")


@functools.partial(
    pl.kernel,
    mesh=_mesh,
    out_type=jax.ShapeDtypeStruct((B_SC, D), jnp.float32),
    scratch_types=[
        pltpu.VMEM((B_PER_W,), jnp.int32),
        pltpu.VMEM((CH, D), jnp.float32),
        pltpu.VMEM((CH, D), jnp.float32),
        pltpu.VMEM_SHARED((V, D), jnp.float32),
        pltpu.SemaphoreType.DMA,
        pltpu.SemaphoreType.DMA,
    ],
    compiler_params=pltpu.CompilerParams(use_tc_tiling_on_sc=False),
)
def _emb(idx_hbm, table_hbm, out_hbm, idx_v, rows0, rows1, table_sh,
         sem0, sem1):
    sid = lax.axis_index("s")
    wid = sid * NC + lax.axis_index("c")
    rows_v = (rows0, rows1)
    sems = (sem0, sem1)

    @pl.when(sid == 0)
    def _():
        pltpu.sync_copy(table_hbm, table_sh)

    base = pl.multiple_of(wid * B_PER_W, 8)
    pltpu.sync_copy(idx_hbm.at[pl.ds(base, B_PER_W)], idx_v)
    plsc.subcore_barrier()

    def fire(ci, b):
        pltpu.async_copy(
            table_sh.at[idx_v.at[pl.ds(ci * CH, CH)]],
            rows_v[b],
            sems[b],
        )

    def drain_and_out(ci, b):
        pltpu.make_async_copy(
            table_sh.at[idx_v.at[pl.ds(ci * CH, CH)]],
            rows_v[b],
            sems[b],
        ).wait()
        pltpu.sync_copy(rows_v[b], out_hbm.at[pl.ds(base + ci * CH, CH)])

    fire(0, 0)

    def pair(g, carry):
        ci0 = 2 * g
        fire(ci0 + 1, 1)
        drain_and_out(ci0, 0)

        @pl.when(g < NPAIR - 1)
        def _():
            fire(ci0 + 2, 0)

        drain_and_out(ci0 + 1, 1)
        return carry

    lax.fori_loop(0, NPAIR, pair, 0)


def _tc_body(idx_ref, table_ref, out_ref):
    idx = idx_ref[...]
    iota = lax.broadcasted_iota(jnp.int32, (BLK, V), 1)
    oh = (idx[:, None] == iota).astype(jnp.bfloat16)
    tab = table_ref[...].astype(jnp.bfloat16)
    out_ref[...] = jax.lax.dot_general(
        oh, tab, (((1,), (0,)), ((), ())),
        preferred_element_type=jnp.float32,
    )


_tc = pl.pallas_call(
    _tc_body,
    grid=(N_BLK,),
    in_specs=[
        pl.BlockSpec((BLK,), lambda i: (i,)),
        pl.BlockSpec((V, D), lambda i: (0, 0)),
    ],
    out_specs=pl.BlockSpec((BLK, D), lambda i: (i, 0)),
    out_shape=jax.ShapeDtypeStruct((B_TC, D), jnp.float32),
)


def kernel(x, table):
    idx = x.astype(jnp.int32).reshape(B)
    out_sc = _emb(idx[:B_SC], table)
    out_tc = _tc(idx[B_SC:], table)
    out = jnp.concatenate([out_sc, out_tc], axis=0)
    return out.reshape(x.shape[0], x.shape[1], D)


# speedup vs baseline: 1.9269x; 1.9269x over previous
import functools

import jax
import jax.numpy as jnp
from jax import lax
from jax.experimental import pallas as pl
from jax.experimental.pallas import tpu as pltpu
from jax.experimental.pallas import tpu_sc as plsc

NC = 2
NS = 16
NW = NC * NS

V = 1440
B = 4096 * 200
D = 64
CH = 160
S_STREAM = 64
S_TEC = CH - S_STREAM
B_PER_W = B // NW
N_CHUNKS = B_PER_W // CH
NPAIR = N_CHUNKS // 2
ROWGRP = 4

_mesh = plsc.VectorSubcoreMesh(core_axis_name="c", subcore_axis_name="s")


@functools.partial(
    pl.kernel,
    mesh=_mesh,
    out_type=jax.ShapeDtypeStruct((B, D), jnp.float32),
    scratch_types=[
        pltpu.VMEM((CH,), jnp.int32),
        pltpu.VMEM((CH,), jnp.int32),
        pltpu.VMEM((CH, D), jnp.float32),
        pltpu.VMEM((CH, D), jnp.float32),
        pltpu.VMEM((V, D), jnp.float32),
        pltpu.VMEM_SHARED((V, D), jnp.float32),
        pltpu.SemaphoreType.DMA,
        pltpu.SemaphoreType.DMA,
        pltpu.SemaphoreType.DMA,
        pltpu.SemaphoreType.DMA,
        pltpu.SemaphoreType.DMA,
        pltpu.SemaphoreType.DMA,
    ],
    compiler_params=pltpu.CompilerParams(use_tc_tiling_on_sc=False),
)
def _emb(idx_hbm, table_hbm, out_hbm, idx0, idx1, rows0, rows1,
         table_v, table_sh, semi0, semi1, semg0, semg1, semo0, semo1):
    sid = lax.axis_index("s")
    wid = sid * NC + lax.axis_index("c")
    base = pl.multiple_of(wid * B_PER_W, 8)
    idx_v = (idx0, idx1)
    rows_v = (rows0, rows1)
    semi = (semi0, semi1)
    semg = (semg0, semg1)
    semo = (semo0, semo1)

    @pl.when(sid == 0)
    def _():
        pltpu.sync_copy(table_hbm, table_sh)

    plsc.subcore_barrier()
    pltpu.sync_copy(table_sh, table_v)

    def idx_copy(ci, b):
        off = pl.multiple_of(base + ci * CH, 8)
        return pltpu.make_async_copy(
            idx_hbm.at[pl.ds(off, CH)], idx_v[b], semi[b]
        )

    def gat_copy(b):
        return pltpu.make_async_copy(
            table_sh.at[idx_v[b].at[pl.ds(0, S_STREAM)]],
            rows_v[b].at[pl.ds(0, S_STREAM)],
            semg[b],
        )

    def out_copy(ci, b):
        return pltpu.make_async_copy(
            rows_v[b], out_hbm.at[pl.ds(base + ci * CH, CH)], semo[b]
        )

    def tec_part(b):
        def body(i, carry):
            iv = idx_v[b][pl.ds(S_STREAM + i * 16, 16)]
            for u in range(0, 16, ROWGRP):
                vals = []
                for w in range(ROWGRP):
                    off = iv[u + w]
                    vals.append(
                        [table_v[off, pl.ds(k * 16, 16)]
                         for k in range(D // 16)]
                    )
                for w in range(ROWGRP):
                    r = S_STREAM + i * 16 + u + w
                    for k in range(D // 16):
                        rows_v[b][r, pl.ds(k * 16, 16)] = vals[w][k]
            return carry

        lax.fori_loop(0, S_TEC // 16, body, 0)

    idx_copy(0, 0).start()
    idx_copy(1, 1).start()

    def step(g, ci, b):
        @pl.when(g >= 1)
        def _():
            out_copy(ci - 2, b).wait()

        idx_copy(ci, b).wait()
        gat_copy(b).start()
        tec_part(b)
        gat_copy(b).wait()

        @pl.when(ci + 2 < N_CHUNKS)
        def _():
            idx_copy(ci + 2, b).start()

        out_copy(ci, b).start()

    def pair(g, carry):
        step(g, 2 * g, 0)
        step(g, 2 * g + 1, 1)
        return carry

    lax.fori_loop(0, NPAIR, pair, 0)
    out_copy(N_CHUNKS - 2, 0).wait()
    out_copy(N_CHUNKS - 1, 1).wait()


def kernel(x, table):
    idx = x.astype(jnp.int32).reshape(B)
    out = _emb(idx, table)
    return out.reshape(x.shape[0], x.shape[1], D)
